# baseline (speedup 1.0000x reference)
import functools

import jax
import jax.numpy as jnp
from jax.experimental import pallas as pl
from jax.experimental.pallas import tpu as pltpu

N = 4096
H = 1024
I = 2816
E_ROUTED = 7
EPAD = 128

BN = 512
BI = 1408
TB = N // BN
IB = I // BI
NE = E_ROUTED + 1


def _router_weights(logits):
    idx = jax.lax.broadcasted_iota(jnp.int32, logits.shape, 1)
    m1 = jnp.max(logits, axis=1, keepdims=True)
    am1 = jnp.min(jnp.where(logits == m1, idx, EPAD), axis=1, keepdims=True)
    mask1 = idx == am1
    l2 = jnp.where(mask1, -1e30, logits)
    m2 = jnp.max(l2, axis=1, keepdims=True)
    am2 = jnp.min(jnp.where(l2 == m2, idx, EPAD), axis=1, keepdims=True)
    mask2 = idx == am2
    e2 = jnp.exp(m2 - m1)
    s1 = 1.0 / (1.0 + e2)
    s2 = e2 * s1
    return jnp.where(mask1, s1, jnp.where(mask2, s2, 0.0))


def _moe_body(x_ref, wr_ref, rb_ref, wg_ref, wu_ref, wd_ref, out_ref, w_scr):
    e = pl.program_id(1)
    ib = pl.program_id(2)

    @pl.when((e == 0) & (ib == 0))
    def _():
        logits = jnp.dot(x_ref[...], wr_ref[...],
                         preferred_element_type=jnp.float32) + rb_ref[...]
        w_scr[...] = _router_weights(logits)

    lane = jax.lax.broadcasted_iota(jnp.int32, (BN, EPAD), 1)
    wsel = jnp.sum(jnp.where(lane == (e - 1), w_scr[...], 0.0),
                   axis=1, keepdims=True)
    wcol = jnp.where(e == 0, 1.0, wsel)

    xb = x_ref[...]
    g = jnp.dot(xb, wg_ref[0], preferred_element_type=jnp.float32)
    u = jnp.dot(xb, wu_ref[0], preferred_element_type=jnp.float32)
    h = (g * jax.nn.sigmoid(g) * u).astype(jnp.bfloat16)
    part = jnp.dot(h, wd_ref[0], preferred_element_type=jnp.float32)
    acc = part * wcol

    @pl.when((e == 0) & (ib == 0))
    def _():
        out_ref[...] = acc

    @pl.when(~((e == 0) & (ib == 0)))
    def _():
        out_ref[...] += acc


@functools.partial(jax.jit, static_argnames=("interpret",))
def _moe(x, WgA, WuA, WdA, Wr_pad, rb_pad, interpret=False):
    return pl.pallas_call(
        _moe_body,
        grid=(TB, NE, IB),
        in_specs=[
            pl.BlockSpec((BN, H), lambda t, e, i: (t, 0)),
            pl.BlockSpec((H, EPAD), lambda t, e, i: (0, 0)),
            pl.BlockSpec((1, EPAD), lambda t, e, i: (0, 0)),
            pl.BlockSpec((1, H, BI), lambda t, e, i: (e, 0, i)),
            pl.BlockSpec((1, H, BI), lambda t, e, i: (e, 0, i)),
            pl.BlockSpec((1, BI, H), lambda t, e, i: (e, i, 0)),
        ],

        out_specs=pl.BlockSpec((BN, H), lambda t, e, i: (t, 0)),
        out_shape=jax.ShapeDtypeStruct((N, H), jnp.float32),
        scratch_shapes=[pltpu.VMEM((BN, EPAD), jnp.float32)],
        compiler_params=pltpu.CompilerParams(
            dimension_semantics=("parallel", "arbitrary", "arbitrary"),
        ),
        interpret=interpret,
    )(x, Wr_pad, rb_pad, WgA, WuA, WdA)


def kernel(x, Wg_s, Wu_s, Wd_s, Wg, Wu, Wd, Wr, rbias):
    bf = jnp.bfloat16
    WgA = jnp.concatenate([Wg_s[None], Wg], axis=0).astype(bf)
    WuA = jnp.concatenate([Wu_s[None], Wu], axis=0).astype(bf)
    WdA = jnp.concatenate([Wd_s[None], Wd], axis=0).astype(bf)
    Wr_pad = jnp.zeros((H, EPAD), bf).at[:, :E_ROUTED].set(Wr.astype(bf))
    rb_pad = jnp.full((1, EPAD), -1e30, jnp.float32).at[0, :E_ROUTED].set(rbias)
    return _moe(x.astype(bf), WgA, WuA, WdA, Wr_pad, rb_pad)
